# baseline (speedup 1.0000x reference)
import functools

import jax
import jax.numpy as jnp
from jax import lax
from jax.experimental import pallas as pl
from jax.experimental.pallas import tpu as pltpu
from jax.experimental.pallas import tpu_sc as plsc

V = 100000
E = 6400000
NC = 2
NS = 16
NW = NC * NS
VPAD = 100352
CHUNK = 12800
NCHUNKS = E // CHUNK
CPW = NCHUNKS // NW
NTAIL = NCHUNKS - CPW * NW
SL = VPAD // NS
VB = VPAD // NW


def _scatter_body(eij_hbm, val_hbm, zeros_hbm, partial_hbm,
                  eij_sh, idx_v0, idx_v1, val_v0, val_v1, acc, lsem, ssem):
    cid = lax.axis_index("c")
    sid = lax.axis_index("s")
    wid = cid * NS + sid
    idx_bufs = (idx_v0, idx_v1)
    val_bufs = (val_v0, val_v1)

    pltpu.sync_copy(zeros_hbm.at[pl.ds(sid * SL, SL)], acc.at[pl.ds(sid * SL, SL)])
    plsc.subcore_barrier()

    def _start(chunk_no, b):
        off = pl.multiple_of(chunk_no * CHUNK, 128)
        d1 = pltpu.async_copy(eij_hbm.at[:, pl.ds(off, CHUNK)],
                              eij_sh.at[sid, b], lsem)
        d2 = pltpu.async_copy(val_hbm.at[pl.ds(off, CHUNK)], val_bufs[b], lsem)
        return d1, d2

    def _wait(ld):
        ld[0].wait()
        ld[1].wait()

    def _compact(b):
        pltpu.sync_copy(eij_sh.at[sid, b, 1], idx_bufs[b])

    def _scatter(b):
        return pltpu.async_copy(val_bufs[b], acc.at[idx_bufs[b]], ssem,
                                add=True)

    def _drain(b):
        pltpu.make_async_copy(val_hbm.at[pl.ds(0, CHUNK)], val_bufs[b],
                              ssem).wait()

    ld = _start(wid * CPW, 0)
    _wait(ld)
    _compact(0)
    for k in range(CPW):
        b = k % 2
        _scatter(b)
        if k + 1 < CPW:
            ld = _start(wid * CPW + k + 1, 1 - b)
            _wait(ld)
            _compact(1 - b)
        _drain(b)
    @pl.when(wid < NTAIL)
    def _tail():
        ld2 = _start(NW * CPW + wid, 0)
        _wait(ld2)
        _compact(0)
        _scatter(0)
        _drain(0)

    plsc.subcore_barrier()
    pltpu.sync_copy(acc.at[pl.ds(sid * SL, SL)],
                    partial_hbm.at[pl.ds(cid * VPAD + sid * SL, SL)])


def _finalize_body(partial_hbm, b_hbm, r_hbm, p0_v, p1_v, b_v, r_v):
    cid = lax.axis_index("c")
    sid = lax.axis_index("s")
    wid = cid * NS + sid
    base = wid * VB

    pltpu.sync_copy(partial_hbm.at[pl.ds(base, VB)], p0_v)
    pltpu.sync_copy(partial_hbm.at[pl.ds(VPAD + base, VB)], p1_v)
    pltpu.sync_copy(b_hbm.at[pl.ds(base, VB)], b_v)

    @pl.loop(0, VB // 16)
    def _rows(i):
        s = pl.ds(i * 16, 16)
        r_v[s] = b_v[s] - (p0_v[s] + p1_v[s])

    pltpu.sync_copy(r_v, r_hbm.at[pl.ds(base, VB)])


def kernel(vertex_attr, edgeij_pair, edge_attr, g, batch):
    del g, batch
    mesh = plsc.VectorSubcoreMesh(core_axis_name="c", subcore_axis_name="s")

    scatter_k = pl.kernel(
        _scatter_body,
        out_type=jax.ShapeDtypeStruct((NC * VPAD,), jnp.float32),
        mesh=mesh,
        scratch_types=[
            pltpu.VMEM_SHARED((NS, 2, 2, CHUNK), jnp.int32),
            pltpu.VMEM((CHUNK,), jnp.int32),
            pltpu.VMEM((CHUNK,), jnp.int32),
            pltpu.VMEM((CHUNK,), jnp.float32),
            pltpu.VMEM((CHUNK,), jnp.float32),
            pltpu.VMEM_SHARED((VPAD,), jnp.float32),
            pltpu.SemaphoreType.DMA,
            pltpu.SemaphoreType.DMA,
        ],
    )
    finalize_k = pl.kernel(
        _finalize_body,
        out_type=jax.ShapeDtypeStruct((VPAD,), jnp.float32),
        mesh=mesh,
        scratch_types=[
            pltpu.VMEM((VB,), jnp.float32),
            pltpu.VMEM((VB,), jnp.float32),
            pltpu.VMEM((VB,), jnp.float32),
            pltpu.VMEM((VB,), jnp.float32),
        ],
    )

    zeros = jnp.zeros((VPAD,), jnp.float32)
    partial = scatter_k(edgeij_pair, edge_attr, zeros)
    b_col = vertex_attr[:, 0]
    x_col = vertex_attr[:, 1]
    b_pad = jnp.pad(b_col, (0, VPAD - V))
    r = finalize_k(partial, b_pad)[:V]
    return jnp.stack([b_col, x_col, r], axis=1)


# speedup vs baseline: 1.0465x; 1.0465x over previous
import functools

import jax
import jax.numpy as jnp
from jax import lax
from jax.experimental import pallas as pl
from jax.experimental.pallas import tpu as pltpu
from jax.experimental.pallas import tpu_sc as plsc

V = 100000
E = 6400000
NC = 2
NS = 16
NW = NC * NS
VPAD = 100352
CHUNK = 12800
NCHUNKS = E // CHUNK
CPW = NCHUNKS // NW
NTAIL = NCHUNKS - CPW * NW
SL = VPAD // NS
VB = VPAD // NW


def _fused_body(eij_hbm, val_hbm, b_hbm, partial_hbm, r_hbm,
                eij_sh, idx_v0, idx_v1, val_v0, val_v1, pb0, pb1, acc,
                lsem, ssem):
    cid = lax.axis_index("c")
    sid = lax.axis_index("s")
    wid = cid * NS + sid
    idx_bufs = (idx_v0, idx_v1)
    val_bufs = (val_v0, val_v1)

    @pl.loop(0, SL // 16, unroll=8)
    def _z(j):
        val_v0[pl.ds(j * 16, 16)] = jnp.zeros((16,), jnp.float32)

    pltpu.sync_copy(val_v0.at[pl.ds(0, SL)], acc.at[pl.ds(sid * SL, SL)])
    plsc.subcore_barrier()

    def _start(chunk_no, b):
        off = pl.multiple_of(chunk_no * CHUNK, 128)
        d1 = pltpu.async_copy(eij_hbm.at[:, pl.ds(off, CHUNK)],
                              eij_sh.at[sid, b], lsem)
        d2 = pltpu.async_copy(val_hbm.at[pl.ds(off, CHUNK)], val_bufs[b], lsem)
        return d1, d2

    def _wait(ld):
        ld[0].wait()
        ld[1].wait()

    def _compact(b):
        pltpu.sync_copy(eij_sh.at[sid, b, 1], idx_bufs[b])

    def _scatter(b):
        return pltpu.async_copy(val_bufs[b], acc.at[idx_bufs[b]], ssem,
                                add=True)

    def _drain(b):
        pltpu.make_async_copy(val_hbm.at[pl.ds(0, CHUNK)], val_bufs[b],
                              ssem).wait()

    ld = _start(wid * CPW, 0)
    _wait(ld)
    _compact(0)
    for k in range(CPW):
        b = k % 2
        _scatter(b)
        if k + 1 < CPW:
            ld = _start(wid * CPW + k + 1, 1 - b)
            _wait(ld)
            _compact(1 - b)
        _drain(b)
    @pl.when(wid < NTAIL)
    def _tail():
        ld2 = _start(NW * CPW + wid, 0)
        _wait(ld2)
        _compact(0)
        _scatter(0)
        _drain(0)

    plsc.subcore_barrier()
    pltpu.sync_copy(acc.at[pl.ds(sid * SL, SL)],
                    partial_hbm.at[pl.ds(cid * VPAD + sid * SL, SL)])
    plsc.subcore_barrier()

    base = wid * VB
    p0_v = pb0
    p1_v = pb1
    b_v = val_v0
    r_v = val_v1
    pltpu.sync_copy(partial_hbm.at[pl.ds(base, VB)], p0_v)
    pltpu.sync_copy(partial_hbm.at[pl.ds(VPAD + base, VB)], p1_v)
    pltpu.sync_copy(b_hbm.at[pl.ds(base, VB)], b_v.at[pl.ds(0, VB)])

    @pl.loop(0, VB // 16)
    def _rows(i):
        s = pl.ds(i * 16, 16)
        r_v[s] = b_v[s] - (p0_v[s] + p1_v[s])

    pltpu.sync_copy(r_v.at[pl.ds(0, VB)], r_hbm.at[pl.ds(base, VB)])


def kernel(vertex_attr, edgeij_pair, edge_attr, g, batch):
    del g, batch
    mesh = plsc.VectorSubcoreMesh(core_axis_name="c", subcore_axis_name="s")

    fused_k = pl.kernel(
        _fused_body,
        out_type=(jax.ShapeDtypeStruct((NC * VPAD,), jnp.float32),
                  jax.ShapeDtypeStruct((VPAD,), jnp.float32)),
        mesh=mesh,
        scratch_types=[
            pltpu.VMEM_SHARED((NS, 2, 2, CHUNK), jnp.int32),
            pltpu.VMEM((CHUNK,), jnp.int32),
            pltpu.VMEM((CHUNK,), jnp.int32),
            pltpu.VMEM((CHUNK,), jnp.float32),
            pltpu.VMEM((CHUNK,), jnp.float32),
            pltpu.VMEM((VB,), jnp.float32),
            pltpu.VMEM((VB,), jnp.float32),
            pltpu.VMEM_SHARED((VPAD,), jnp.float32),
            pltpu.SemaphoreType.DMA,
            pltpu.SemaphoreType.DMA,
        ],
    )

    b_col = vertex_attr[:, 0]
    x_col = vertex_attr[:, 1]
    b_pad = jnp.pad(b_col, (0, VPAD - V))
    _, r = fused_k(edgeij_pair, edge_attr, b_pad)
    return jnp.stack([b_col, x_col, r[:V]], axis=1)
